# baseline (speedup 1.0000x reference)
import functools

import numpy as np

import jax
import jax.numpy as jnp
from jax import lax
from jax.experimental import pallas as pl
from jax.experimental.pallas import tpu as pltpu
from jax.experimental.pallas import tpu_sc as plsc

L = 16
B = 512
SUB = 128
NSUB2 = 2 * B // SUB
GROUPS = B // L

_CP = pltpu.CompilerParams(
    needs_layout_passes=False, use_tc_tiling_on_sc=False)


def _iota16():
    return lax.iota(jnp.int32, L)


@functools.lru_cache(maxsize=None)
def _build_relayout(R, nc, ns):
    NW = nc * ns
    nfaces = 6
    units = nfaces * (R // 8)
    assert units % NW == 0
    upw = units // NW
    assert upw % 2 == 0
    ntc = R // 128
    unit_floats = 8 * R * 3
    tile_floats = 8 * R
    plane = R * R
    V = nfaces * plane * 3 // 8
    mesh = plsc.VectorSubcoreMesh(
        core_axis_name="c", subcore_axis_name="s",
        num_cores=nc, num_subcores=ns)

    io = np.arange(16)

    def body(src_hbm, tab_hbm, bufs0, bufs1, ob0, ob1, isem0, isem1,
             osem0, osem1):
        wid = lax.axis_index("s") * nc + lax.axis_index("c")
        bufs = (bufs0, bufs1)
        obs = (ob0, ob1)
        isems = (isem0, isem1)
        osems = (osem0, osem1)

        def unit_ft(j):
            u = wid * upw + j
            return lax.shift_right_logical(u, 7), u & 127

        def src_off(f, c, tr):
            return pl.multiple_of(
                (f * 3 + c) * plane + tr * tile_floats, tile_floats)

        def in_copies(j, s):
            f, tr = unit_ft(j)
            return [
                pltpu.make_async_copy(
                    src_hbm.at[pl.ds(src_off(f, c, tr), tile_floats)],
                    bufs[s].at[c], isems[s])
                for c in range(3)
            ]

        def out_copy(j, s):
            f, tr = unit_ft(j)
            row0 = (f * R + tr * 8) * (R * 3 // 8)
            return pltpu.make_async_copy(
                obs[s], tab_hbm.at[pl.ds(row0, unit_floats // 8)], osems[s])

        def interleave(s):
            buf = bufs[s]
            ob = obs[s]
            io3 = _iota16() * 3
            consts = [
                (lax.shift_right_logical(io3 + c, 3), (io3 + c) & 7)
                for c in range(3)
            ]

            @pl.loop(0, 8 * ntc)
            def _m(m):
                r = lax.shift_right_logical(m, 3)
                tc = m & (ntc - 1)
                rbase = r * (R * 3 // 8) + tc * 48
                sbase = tc * 1024 + r * 128
                for c in range(3):
                    rowc, colc = consts[c]
                    for k in range(8):
                        sv = buf[c, pl.ds(sbase + k * 16, L)]
                        plsc.store_scatter(
                            ob, [rowc + (rbase + 6 * k), colc], sv)

        for d in in_copies(0, 0):
            d.start()

        @pl.loop(0, upw, step=2)
        def _unit(jj):
            for s in (0, 1):
                j = jj + s
                for d in in_copies(j, s):
                    d.wait()

                @pl.when(j + 1 < upw)
                def _():
                    for d in in_copies(j + 1, s ^ 1):
                        d.start()

                @pl.when(j >= 2)
                def _():
                    out_copy(j - 2, s).wait()

                interleave(s)
                out_copy(j, s).start()

        out_copy(upw - 2, 0).wait()
        out_copy(upw - 1, 1).wait()

    return pl.kernel(
        body,
        out_type=jax.ShapeDtypeStruct((V, 8), jnp.float32),
        mesh=mesh,
        compiler_params=_CP,
        scratch_types=[
            pltpu.VMEM((3, tile_floats), jnp.float32),
            pltpu.VMEM((3, tile_floats), jnp.float32),
            pltpu.VMEM((unit_floats // 8, 8), jnp.float32),
            pltpu.VMEM((unit_floats // 8, 8), jnp.float32),
            pltpu.SemaphoreType.DMA,
            pltpu.SemaphoreType.DMA,
            pltpu.SemaphoreType.DMA,
            pltpu.SemaphoreType.DMA,
        ],
    )


@functools.lru_cache(maxsize=None)
def _build_main(N, R, nc, ns):
    NW = nc * ns
    assert N % (NW * B) == 0, (N, NW, B)
    per_w = N // NW
    chunks = per_w // B
    assert chunks % 2 == 0
    plane = 512 * 512
    nrows8 = 6 * R * R * 3 // 8
    mesh = plsc.VectorSubcoreMesh(
        core_axis_name="c", subcore_axis_name="s",
        num_cores=nc, num_subcores=ns)

    def body(vd_hbm, tab_hbm, out_hbm, *refs):
        (vb0, vb1, wi0a, wi0b, wi1a, wi1b, qa0, qa1, wts0, wts1,
         win0a, win0b, win1a, win1b, ob0, ob1,
         isem0, isem1, gsem0, gsem1, osem0, osem1) = refs
        vbs = (vb0, vb1)
        wi0s = (wi0a, wi0b)
        wi1s = (wi1a, wi1b)
        qs = (qa0, qa1)
        wtss = (wts0, wts1)
        win0s = (win0a, win0b)
        win1s = (win1a, win1b)
        obs = (ob0, ob1)
        isems = (isem0, isem1)
        gsems = (gsem0, gsem1)
        osems = (osem0, osem1)
        wid = lax.axis_index("s") * nc + lax.axis_index("c")
        iota = _iota16()

        def voff(g):
            s0 = wid * per_w + g * B
            img = lax.shift_right_logical(s0, 18)
            q = s0 & (plane - 1)
            return pl.multiple_of(img * (3 * plane) + q, B)

        def in_copies(g, s):
            vo = voff(g)
            return [
                pltpu.make_async_copy(
                    vd_hbm.at[pl.ds(vo + c * plane, B)],
                    vbs[s].at[c], isems[s])
                for c in range(3)
            ]

        def out_copies(g, s):
            vo = voff(g)
            return [
                pltpu.make_async_copy(
                    obs[s].at[c],
                    out_hbm.at[pl.ds(vo + c * plane, B)], osems[s])
                for c in range(3)
            ]

        def gather_copies(g, s):
            ds_ = []
            for j in range(NSUB2):
                ds_.append(pltpu.make_async_copy(
                    tab_hbm.at[wi0s[s].at[j]],
                    win0s[s].at[pl.ds(j * SUB, SUB)], gsems[s]))
                ds_.append(pltpu.make_async_copy(
                    tab_hbm.at[wi1s[s].at[j]],
                    win1s[s].at[pl.ds(j * SUB, SUB)], gsems[s]))
            return ds_

        def compute(s):
            vb = vbs[s]
            wi0 = wi0s[s]
            wi1 = wi1s[s]
            qr = qs[s]
            wts = wtss[s]

            @pl.loop(0, GROUPS, unroll=4)
            def _grp(i):
                sl = pl.ds(i * L, L)
                vx = vb[0, sl]
                vy = vb[1, sl]
                vz = vb[2, sl]
                gx, gy, gz = vx, vz, -vy
                ax, ay, az = jnp.abs(gx), jnp.abs(gy), jnp.abs(gz)
                is_x = (ax >= ay) & (ax >= az)
                is_y = (~is_x) & (ay >= az)
                ma = jnp.where(is_x, ax, jnp.where(is_y, ay, az))
                ngz = -gz
                s1 = jnp.where(gx >= 0, ngz, gz)
                s2 = jnp.where(gz >= 0, gx, -gx)
                sc = jnp.where(is_x, s1, jnp.where(is_y, gx, s2))
                t1 = jnp.where(gy >= 0, gz, ngz)
                tc = jnp.where(is_y, t1, -gy)
                face = jnp.where(
                    is_x, jnp.where(gx >= 0, 0, 1),
                    jnp.where(is_y, jnp.where(gy >= 0, 2, 3),
                              jnp.where(gz >= 0, 4, 5)))
                inv = 1.0 / jnp.maximum(ma, 1e-12)
                half = R * 0.5
                xf = (sc * inv) * half + (half - 0.5)
                yf = (tc * inv) * half + (half - 0.5)
                xt = xf.astype(jnp.int32)
                yt = yf.astype(jnp.int32)
                xtf = xt.astype(jnp.float32)
                ytf = yt.astype(jnp.float32)
                xneg = xf < xtf
                yneg = yf < ytf
                wx = xf - (xtf - jnp.where(xneg, 1.0, 0.0))
                wy = yf - (ytf - jnp.where(yneg, 1.0, 0.0))
                x0 = jnp.maximum(xt - jnp.where(xneg, 1, 0), 0)
                y0 = jnp.maximum(yt - jnp.where(yneg, 1, 0), 0)
                x1 = jnp.minimum(x0 + 1, R - 1)
                y1 = jnp.minimum(y0 + 1, R - 1)
                fb = face << 20
                a0 = (fb + (y0 << 10) + x0) * 3
                a2 = (fb + (y1 << 10) + x0) * 3
                dx3 = (x1 - x0) * 3
                k0 = lax.shift_right_logical(a0, 3)
                k2 = lax.shift_right_logical(a2, 3)
                o0 = a0 - (k0 << 3)
                o2 = a2 - (k2 << 3)
                rows = i * L + iota
                jpos = rows << 4
                qr[0, sl] = jpos + o0
                qr[1, sl] = jpos + o0 + dx3
                qr[2, sl] = jpos + o2
                qr[3, sl] = jpos + o2 + dx3
                wts[0, sl] = wx
                wts[1, sl] = wy
                two_j = rows << 1
                er = lax.shift_right_logical(two_j, 7)
                ec = two_j & 127
                k0b = jnp.minimum(k0 + 1, nrows8 - 1)
                k2b = jnp.minimum(k2 + 1, nrows8 - 1)
                plsc.store_scatter(wi0, [er, ec], k0)
                plsc.store_scatter(wi0, [er, ec + 1], k0b)
                plsc.store_scatter(wi1, [er, ec], k2)
                plsc.store_scatter(wi1, [er, ec + 1], k2b)

        def combine(s):
            qr = qs[s]
            wts = wtss[s]
            win0 = win0s[s]
            win1 = win1s[s]
            ob = obs[s]

            @pl.loop(0, GROUPS, unroll=4)
            def _cmb(i):
                sl = pl.ds(i * L, L)
                qa, qb, qc, qd = qr[0, sl], qr[1, sl], qr[2, sl], qr[3, sl]
                wx = wts[0, sl]
                wy = wts[1, sl]
                for ch in range(3):
                    pa = qa + ch
                    pb = qb + ch
                    pc = qc + ch
                    pd = qd + ch
                    c00 = plsc.load_gather(
                        win0, [lax.shift_right_logical(pa, 3), pa & 7])
                    c01 = plsc.load_gather(
                        win0, [lax.shift_right_logical(pb, 3), pb & 7])
                    c10 = plsc.load_gather(
                        win1, [lax.shift_right_logical(pc, 3), pc & 7])
                    c11 = plsc.load_gather(
                        win1, [lax.shift_right_logical(pd, 3), pd & 7])
                    a = c00 + wx * (c01 - c00)
                    b2 = c10 + wx * (c11 - c10)
                    ob[ch, sl] = a + wy * (b2 - a)

        for d in in_copies(0, 0):
            d.start()

        @pl.loop(0, chunks, step=2)
        def _chunk(gg):
            for s in (0, 1):
                g = gg + s
                for d in in_copies(g, s):
                    d.wait()
                compute(s)
                for d in gather_copies(g, s):
                    d.start()

                @pl.when(g + 1 < chunks)
                def _():
                    for d in in_copies(g + 1, s ^ 1):
                        d.start()

                @pl.when(g >= 1)
                def _():
                    for d in gather_copies(g - 1, s ^ 1):
                        d.wait()

                    @pl.when(g >= 3)
                    def _():
                        for d in out_copies(g - 3, s ^ 1):
                            d.wait()

                    combine(s ^ 1)
                    for d in out_copies(g - 1, s ^ 1):
                        d.start()

        for d in gather_copies(chunks - 1, 1):
            d.wait()
        for d in out_copies(chunks - 3, 1):
            d.wait()
        combine(1)
        for d in out_copies(chunks - 1, 1):
            d.start()
        for d in out_copies(chunks - 2, 0):
            d.wait()
        for d in out_copies(chunks - 1, 1):
            d.wait()

    return pl.kernel(
        body,
        out_type=jax.ShapeDtypeStruct((N * 3,), jnp.float32),
        mesh=mesh,
        compiler_params=_CP,
        scratch_types=(
            [pltpu.VMEM((3, B), jnp.float32)] * 2 +
            [pltpu.VMEM((NSUB2, SUB), jnp.int32)] * 4 +
            [pltpu.VMEM((4, B), jnp.int32)] * 2 +
            [pltpu.VMEM((2, B), jnp.float32)] * 2 +
            [pltpu.VMEM((2 * B, 8), jnp.float32)] * 4 +
            [pltpu.VMEM((3, B), jnp.float32)] * 2 +
            [pltpu.SemaphoreType.DMA] * 6
        ),
    )


def _tiled_view_flat(x):
    A, R0, R1 = x.shape[0], x.shape[1], x.shape[2]
    xp = x.transpose(0, 3, 1, 2)
    x6 = xp.reshape(A, 3, R0 // 8, 8, R1 // 128, 128)
    x6 = x6.transpose(0, 1, 2, 4, 3, 5)
    return x6.reshape(-1)


def kernel(viewdirs, base):
    prefix = viewdirs.shape[:-1]
    nimg, H, W = prefix
    assert (H, W) == (512, 512) and base.shape[1] == 1024
    N = nimg * H * W
    R = base.shape[1]
    info = plsc.get_sparse_core_info()
    nc, ns = info.num_cores, info.num_subcores

    base_flat = _tiled_view_flat(base)
    vd_flat = _tiled_view_flat(viewdirs)
    tab = _build_relayout(R, nc, ns)(base_flat)
    outf = _build_main(N, R, nc, ns)(vd_flat, tab)
    out = outf.reshape(nimg, 3, H // 8, W // 128, 8, 128)
    out = out.transpose(0, 1, 2, 4, 3, 5).reshape(nimg, 3, H, W)
    return out.transpose(0, 2, 3, 1)


# speedup vs baseline: 1.0580x; 1.0580x over previous
import functools

import numpy as np

import jax
import jax.numpy as jnp
from jax import lax
from jax.experimental import pallas as pl
from jax.experimental.pallas import tpu as pltpu
from jax.experimental.pallas import tpu_sc as plsc

L = 16
B = 1024
SUB = 128
NSUB2 = 2 * B // SUB
GROUPS = B // L

_CP = pltpu.CompilerParams(
    needs_layout_passes=False, use_tc_tiling_on_sc=False)


def _iota16():
    return lax.iota(jnp.int32, L)


@functools.lru_cache(maxsize=None)
def _build_relayout(R, nc, ns):
    NW = nc * ns
    nfaces = 6
    units = nfaces * (R // 8)
    assert units % NW == 0
    upw = units // NW
    assert upw % 2 == 0
    ntc = R // 128
    unit_floats = 8 * R * 3
    tile_floats = 8 * R
    plane = R * R
    V = nfaces * plane * 3 // 8
    mesh = plsc.VectorSubcoreMesh(
        core_axis_name="c", subcore_axis_name="s",
        num_cores=nc, num_subcores=ns)

    io = np.arange(16)

    def body(src_hbm, tab_hbm, bufs0, bufs1, ob0, ob1, isem0, isem1,
             osem0, osem1):
        wid = lax.axis_index("s") * nc + lax.axis_index("c")
        bufs = (bufs0, bufs1)
        obs = (ob0, ob1)
        isems = (isem0, isem1)
        osems = (osem0, osem1)

        def unit_ft(j):
            u = wid * upw + j
            return lax.shift_right_logical(u, 7), u & 127

        def src_off(f, c, tr):
            return pl.multiple_of(
                (f * 3 + c) * plane + tr * tile_floats, tile_floats)

        def in_copies(j, s):
            f, tr = unit_ft(j)
            return [
                pltpu.make_async_copy(
                    src_hbm.at[pl.ds(src_off(f, c, tr), tile_floats)],
                    bufs[s].at[c], isems[s])
                for c in range(3)
            ]

        def out_copy(j, s):
            f, tr = unit_ft(j)
            row0 = (f * R + tr * 8) * (R * 3 // 8)
            return pltpu.make_async_copy(
                obs[s], tab_hbm.at[pl.ds(row0, unit_floats // 8)], osems[s])

        def interleave(s):
            buf = bufs[s]
            ob = obs[s]
            io3 = _iota16() * 3
            consts = [
                (lax.shift_right_logical(io3 + c, 3), (io3 + c) & 7)
                for c in range(3)
            ]

            @pl.loop(0, 8 * ntc)
            def _m(m):
                r = lax.shift_right_logical(m, 3)
                tc = m & (ntc - 1)
                rbase = r * (R * 3 // 8) + tc * 48
                sbase = tc * 1024 + r * 128
                for c in range(3):
                    rowc, colc = consts[c]
                    for k in range(8):
                        sv = buf[c, pl.ds(sbase + k * 16, L)]
                        plsc.store_scatter(
                            ob, [rowc + (rbase + 6 * k), colc], sv)

        for d in in_copies(0, 0):
            d.start()

        @pl.loop(0, upw, step=2)
        def _unit(jj):
            for s in (0, 1):
                j = jj + s
                for d in in_copies(j, s):
                    d.wait()

                @pl.when(j + 1 < upw)
                def _():
                    for d in in_copies(j + 1, s ^ 1):
                        d.start()

                @pl.when(j >= 2)
                def _():
                    out_copy(j - 2, s).wait()

                interleave(s)
                out_copy(j, s).start()

        out_copy(upw - 2, 0).wait()
        out_copy(upw - 1, 1).wait()

    return pl.kernel(
        body,
        out_type=jax.ShapeDtypeStruct((V, 8), jnp.float32),
        mesh=mesh,
        compiler_params=_CP,
        scratch_types=[
            pltpu.VMEM((3, tile_floats), jnp.float32),
            pltpu.VMEM((3, tile_floats), jnp.float32),
            pltpu.VMEM((unit_floats // 8, 8), jnp.float32),
            pltpu.VMEM((unit_floats // 8, 8), jnp.float32),
            pltpu.SemaphoreType.DMA,
            pltpu.SemaphoreType.DMA,
            pltpu.SemaphoreType.DMA,
            pltpu.SemaphoreType.DMA,
        ],
    )


@functools.lru_cache(maxsize=None)
def _build_main(N, R, nc, ns):
    NW = nc * ns
    assert N % (NW * B) == 0, (N, NW, B)
    per_w = N // NW
    chunks = per_w // B
    assert chunks % 2 == 0
    plane = 512 * 512
    nrows8 = 6 * R * R * 3 // 8
    mesh = plsc.VectorSubcoreMesh(
        core_axis_name="c", subcore_axis_name="s",
        num_cores=nc, num_subcores=ns)

    def body(vd_hbm, tab_hbm, out_hbm, *refs):
        (vb0, vb1, wi0a, wi0b, wi1a, wi1b, qa0, qa1, wts0, wts1,
         win0a, win0b, win1a, win1b, ob0, ob1,
         isem0, isem1, gsem0, gsem1, osem0, osem1) = refs
        vbs = (vb0, vb1)
        wi0s = (wi0a, wi0b)
        wi1s = (wi1a, wi1b)
        qs = (qa0, qa1)
        wtss = (wts0, wts1)
        win0s = (win0a, win0b)
        win1s = (win1a, win1b)
        obs = (ob0, ob1)
        isems = (isem0, isem1)
        gsems = (gsem0, gsem1)
        osems = (osem0, osem1)
        wid = lax.axis_index("s") * nc + lax.axis_index("c")
        iota = _iota16()

        def voff(g):
            s0 = wid * per_w + g * B
            img = lax.shift_right_logical(s0, 18)
            q = s0 & (plane - 1)
            return pl.multiple_of(img * (3 * plane) + q, B)

        def in_copies(g, s):
            vo = voff(g)
            return [
                pltpu.make_async_copy(
                    vd_hbm.at[pl.ds(vo + c * plane, B)],
                    vbs[s].at[c], isems[s])
                for c in range(3)
            ]

        def out_copies(g, s):
            vo = voff(g)
            return [
                pltpu.make_async_copy(
                    obs[s].at[c],
                    out_hbm.at[pl.ds(vo + c * plane, B)], osems[s])
                for c in range(3)
            ]

        def gather_copies(g, s):
            ds_ = []
            for j in range(NSUB2):
                ds_.append(pltpu.make_async_copy(
                    tab_hbm.at[wi0s[s].at[j]],
                    win0s[s].at[pl.ds(j * SUB, SUB)], gsems[s]))
                ds_.append(pltpu.make_async_copy(
                    tab_hbm.at[wi1s[s].at[j]],
                    win1s[s].at[pl.ds(j * SUB, SUB)], gsems[s]))
            return ds_

        def compute(s):
            vb = vbs[s]
            wi0 = wi0s[s]
            wi1 = wi1s[s]
            qr = qs[s]
            wts = wtss[s]

            @pl.loop(0, GROUPS)
            def _grp(i):
                sl = pl.ds(i * L, L)
                vx = vb[0, sl]
                vy = vb[1, sl]
                vz = vb[2, sl]
                gx, gy, gz = vx, vz, -vy
                ax, ay, az = jnp.abs(gx), jnp.abs(gy), jnp.abs(gz)
                is_x = (ax >= ay) & (ax >= az)
                is_y = (~is_x) & (ay >= az)
                ma = jnp.where(is_x, ax, jnp.where(is_y, ay, az))
                ngz = -gz
                s1 = jnp.where(gx >= 0, ngz, gz)
                s2 = jnp.where(gz >= 0, gx, -gx)
                sc = jnp.where(is_x, s1, jnp.where(is_y, gx, s2))
                t1 = jnp.where(gy >= 0, gz, ngz)
                tc = jnp.where(is_y, t1, -gy)
                face = jnp.where(
                    is_x, jnp.where(gx >= 0, 0, 1),
                    jnp.where(is_y, jnp.where(gy >= 0, 2, 3),
                              jnp.where(gz >= 0, 4, 5)))
                inv = 1.0 / jnp.maximum(ma, 1e-12)
                half = R * 0.5
                xf = (sc * inv) * half + (half - 0.5)
                yf = (tc * inv) * half + (half - 0.5)
                xt = xf.astype(jnp.int32)
                yt = yf.astype(jnp.int32)
                xtf = xt.astype(jnp.float32)
                ytf = yt.astype(jnp.float32)
                xneg = xf < xtf
                yneg = yf < ytf
                wx = xf - (xtf - jnp.where(xneg, 1.0, 0.0))
                wy = yf - (ytf - jnp.where(yneg, 1.0, 0.0))
                x0 = jnp.maximum(xt - jnp.where(xneg, 1, 0), 0)
                y0 = jnp.maximum(yt - jnp.where(yneg, 1, 0), 0)
                x1 = jnp.minimum(x0 + 1, R - 1)
                y1 = jnp.minimum(y0 + 1, R - 1)
                fb = face << 20
                a0 = (fb + (y0 << 10) + x0) * 3
                a2 = (fb + (y1 << 10) + x0) * 3
                dx3 = (x1 - x0) * 3
                k0 = lax.shift_right_logical(a0, 3)
                k2 = lax.shift_right_logical(a2, 3)
                o0 = a0 - (k0 << 3)
                o2 = a2 - (k2 << 3)
                rows = i * L + iota
                jpos = rows << 4
                qr[0, sl] = jpos + o0
                qr[1, sl] = jpos + o0 + dx3
                qr[2, sl] = jpos + o2
                qr[3, sl] = jpos + o2 + dx3
                wts[0, sl] = wx
                wts[1, sl] = wy
                two_j = rows << 1
                er = lax.shift_right_logical(two_j, 7)
                ec = two_j & 127
                k0b = jnp.minimum(k0 + 1, nrows8 - 1)
                k2b = jnp.minimum(k2 + 1, nrows8 - 1)
                plsc.store_scatter(wi0, [er, ec], k0)
                plsc.store_scatter(wi0, [er, ec + 1], k0b)
                plsc.store_scatter(wi1, [er, ec], k2)
                plsc.store_scatter(wi1, [er, ec + 1], k2b)

        def combine(s):
            qr = qs[s]
            wts = wtss[s]
            win0 = win0s[s]
            win1 = win1s[s]
            ob = obs[s]

            @pl.loop(0, GROUPS)
            def _cmb(i):
                sl = pl.ds(i * L, L)
                qa, qb, qc, qd = qr[0, sl], qr[1, sl], qr[2, sl], qr[3, sl]
                wx = wts[0, sl]
                wy = wts[1, sl]
                for ch in range(3):
                    pa = qa + ch
                    pb = qb + ch
                    pc = qc + ch
                    pd = qd + ch
                    c00 = plsc.load_gather(
                        win0, [lax.shift_right_logical(pa, 3), pa & 7])
                    c01 = plsc.load_gather(
                        win0, [lax.shift_right_logical(pb, 3), pb & 7])
                    c10 = plsc.load_gather(
                        win1, [lax.shift_right_logical(pc, 3), pc & 7])
                    c11 = plsc.load_gather(
                        win1, [lax.shift_right_logical(pd, 3), pd & 7])
                    a = c00 + wx * (c01 - c00)
                    b2 = c10 + wx * (c11 - c10)
                    ob[ch, sl] = a + wy * (b2 - a)

        for d in in_copies(0, 0):
            d.start()

        @pl.loop(0, chunks, step=2)
        def _chunk(gg):
            for s in (0, 1):
                g = gg + s
                for d in in_copies(g, s):
                    d.wait()
                compute(s)
                for d in gather_copies(g, s):
                    d.start()

                @pl.when(g + 1 < chunks)
                def _():
                    for d in in_copies(g + 1, s ^ 1):
                        d.start()

                @pl.when(g >= 1)
                def _():
                    for d in gather_copies(g - 1, s ^ 1):
                        d.wait()

                    @pl.when(g >= 3)
                    def _():
                        for d in out_copies(g - 3, s ^ 1):
                            d.wait()

                    combine(s ^ 1)
                    for d in out_copies(g - 1, s ^ 1):
                        d.start()

        for d in gather_copies(chunks - 1, 1):
            d.wait()
        for d in out_copies(chunks - 3, 1):
            d.wait()
        combine(1)
        for d in out_copies(chunks - 1, 1):
            d.start()
        for d in out_copies(chunks - 2, 0):
            d.wait()
        for d in out_copies(chunks - 1, 1):
            d.wait()

    return pl.kernel(
        body,
        out_type=jax.ShapeDtypeStruct((N * 3,), jnp.float32),
        mesh=mesh,
        compiler_params=_CP,
        scratch_types=(
            [pltpu.VMEM((3, B), jnp.float32)] * 2 +
            [pltpu.VMEM((NSUB2, SUB), jnp.int32)] * 4 +
            [pltpu.VMEM((4, B), jnp.int32)] * 2 +
            [pltpu.VMEM((2, B), jnp.float32)] * 2 +
            [pltpu.VMEM((2 * B, 8), jnp.float32)] * 4 +
            [pltpu.VMEM((3, B), jnp.float32)] * 2 +
            [pltpu.SemaphoreType.DMA] * 6
        ),
    )


def _tiled_view_flat(x):
    A, R0, R1 = x.shape[0], x.shape[1], x.shape[2]
    xp = x.transpose(0, 3, 1, 2)
    x6 = xp.reshape(A, 3, R0 // 8, 8, R1 // 128, 128)
    x6 = x6.transpose(0, 1, 2, 4, 3, 5)
    return x6.reshape(-1)


def kernel(viewdirs, base):
    prefix = viewdirs.shape[:-1]
    nimg, H, W = prefix
    assert (H, W) == (512, 512) and base.shape[1] == 1024
    N = nimg * H * W
    R = base.shape[1]
    info = plsc.get_sparse_core_info()
    nc, ns = info.num_cores, info.num_subcores

    base_flat = _tiled_view_flat(base)
    vd_flat = _tiled_view_flat(viewdirs)
    tab = _build_relayout(R, nc, ns)(base_flat)
    outf = _build_main(N, R, nc, ns)(vd_flat, tab)
    out = outf.reshape(nimg, 3, H // 8, W // 128, 8, 128)
    out = out.transpose(0, 1, 2, 4, 3, 5).reshape(nimg, 3, H, W)
    return out.transpose(0, 2, 3, 1)
